# baseline (speedup 1.0000x reference)
import functools

import jax
import jax.numpy as jnp
from jax import lax
from jax.experimental import pallas as pl
from jax.experimental.pallas import tpu as pltpu
from jax.experimental.pallas import tpu_sc as plsc

DIM = 32
W = 256
TPW = 2
NW = 32
LANES = 16
GROUPS = W // LANES
VCOV = 999936
QSZ = VCOV // 4
BLK = 3968


def _sc_gather(w128, tail, xp_flat, n, b, t):
    mesh = plsc.VectorSubcoreMesh(core_axis_name="core",
                                  subcore_axis_name="subcore")
    n_win = n // W
    per_w = n_win // NW
    n_tail = tail.shape[0]

    @functools.partial(
        pl.kernel,
        out_type=jax.ShapeDtypeStruct((t, DIM, b), jnp.float32),
        mesh=mesh,
        compiler_params=pltpu.CompilerParams(
            needs_layout_passes=False, disable_bounds_checks=True),
        scratch_types=[
            pltpu.VMEM((per_w * W,), jnp.int32),
            pltpu.VMEM((n_tail, DIM), jnp.float32),
            pltpu.VMEM((W,), jnp.int32),
            pltpu.VMEM((W,), jnp.int32),
            pltpu.VMEM((W, 128), jnp.float32),
            pltpu.VMEM((W, 128), jnp.float32),
            pltpu.VMEM((TPW, DIM, 128), jnp.float32),
            pltpu.VMEM((TPW, DIM, 128), jnp.float32),
            pltpu.SemaphoreType.DMA,
            pltpu.SemaphoreType.DMA,
            pltpu.SemaphoreType.DMA,
            pltpu.SemaphoreType.DMA,
        ],
    )
    def gather_kernel(w_hbm, tail_hbm, x_hbm, o_hbm, idx_all, tail_v,
                      q0, q1, g0, g1, t0b, t1b, gs0, gs1, os0, os1):
        wid = lax.axis_index("subcore") * 2 + lax.axis_index("core")
        base_win = wid * per_w
        pltpu.sync_copy(x_hbm.at[pl.ds(base_win * W, per_w * W)], idx_all)
        pltpu.sync_copy(tail_hbm, tail_v)

        qbufs = (q0, q1)
        gbufs = (g0, g1)
        tbufs = (t0b, t1b)
        gsems = (gs0, gs1)
        osems = (os0, os1)
        rows = [lax.iota(jnp.int32, LANES) + g * LANES
                for g in range(GROUPS)]

        def out_slice(k):
            win = base_win + k
            tt = win // 128
            bt = lax.rem(win, 128) // 4
            t0 = tt * 8 + lax.rem(win, 4) * TPW
            return o_hbm.at[pl.ds(t0, TPW), :, pl.ds(bt * 128, 128)]

        def prep_gather(k, p):
            @pl.loop(0, GROUPS)
            def _(j):
                xg = idx_all[pl.ds(k * W + j * LANES, LANES)]
                q = jnp.minimum(xg // QSZ, 3)
                qbufs[p][pl.ds(j * LANES, LANES)] = jnp.minimum(
                    xg - q * QSZ, QSZ - 1)

            pltpu.async_copy(w_hbm.at[qbufs[p]], gbufs[p], gsems[p])

        def wait_gather(p):
            pltpu.make_async_copy(w_hbm.at[qbufs[p]], gbufs[p],
                                  gsems[p]).wait()

        def start_out(k, p):
            pltpu.async_copy(tbufs[p], out_slice(k), osems[p])

        def wait_out(k, p):
            pltpu.make_async_copy(tbufs[p], out_slice(k), osems[p]).wait()

        def extract(k, p):
            gb = gbufs[p]
            tb = tbufs[p]
            cols, trows, tmask = [], [], []
            for g in range(GROUPS):
                xg = idx_all[pl.ds(k * W + g * LANES, LANES)]
                cols.append(jnp.minimum(xg // QSZ, 3) * DIM)
                trows.append(jnp.maximum(xg - VCOV, 0))
                tmask.append(xg >= VCOV)

            @pl.loop(0, DIM)
            def _(d):
                dvec = jnp.full((LANES,), d, dtype=jnp.int32)
                vals = [plsc.load_gather(gb, [rows[g], cols[g] + d])
                        for g in range(GROUPS)]
                tvals = [plsc.load_gather(tail_v, [trows[g], dvec])
                         for g in range(GROUPS)]
                for g in range(GROUPS):
                    s, j = divmod(g, 128 // LANES)
                    tb[s, d, pl.ds(j * LANES, LANES)] = jnp.where(
                        tmask[g], tvals[g], vals[g])

        prep_gather(0, 0)

        @pl.loop(0, per_w, step=2)
        def _(k):
            for h in range(2):
                kk = k + h

                @pl.when(kk + 1 < per_w)
                def _():
                    prep_gather(kk + 1, (h + 1) % 2)

                wait_gather(h)

                @pl.when(kk >= 2)
                def _():
                    wait_out(kk - 2, h)

                extract(kk, h)
                start_out(kk, h)

        wait_out(per_w - 2, 0)
        wait_out(per_w - 1, 1)

    return gather_kernel(w128, tail, xp_flat)


def _tc_relayout(wt):
    nblk = QSZ // BLK

    def body(w_hbm, o_ref, wbuf, sem):
        i = pl.program_id(0)
        off = pl.multiple_of(i * BLK, 128)
        copies = [
            pltpu.make_async_copy(
                w_hbm.at[:, pl.ds(q * QSZ + off, BLK)], wbuf.at[q], sem)
            for q in range(4)
        ]
        for c in copies:
            c.start()
        for c in copies:
            c.wait()
        for q in range(4):
            o_ref[:, pl.ds(q * DIM, DIM)] = wbuf[q].T

    return pl.pallas_call(
        body,
        grid=(nblk,),
        in_specs=[pl.BlockSpec(memory_space=pltpu.MemorySpace.HBM)],
        out_specs=pl.BlockSpec((BLK, 128), lambda i: (i, 0)),
        out_shape=jax.ShapeDtypeStruct((QSZ, 128), jnp.float32),
        scratch_shapes=[pltpu.VMEM((4, DIM, BLK), jnp.float32),
                        pltpu.SemaphoreType.DMA],
    )(wt)


def kernel(x, weight):
    b, t = x.shape
    n = x.size
    v = weight.shape[0]
    xp = x.reshape(b // 128, 128, t // 8, 8).transpose((2, 0, 3, 1))
    xp_flat = xp.reshape((n,)).astype(jnp.int32)
    w128 = _tc_relayout(weight.T)
    tail = lax.slice(weight, (VCOV, 0), (v, DIM))
    out3 = _sc_gather(w128, tail, xp_flat, n, b, t)
    return out3.transpose((2, 0, 1))


# speedup vs baseline: 1.3318x; 1.3318x over previous
import functools

import jax
import jax.numpy as jnp
from jax import lax
from jax.experimental import pallas as pl
from jax.experimental.pallas import tpu as pltpu
from jax.experimental.pallas import tpu_sc as plsc

DIM = 32
W = 256
TPW = 2
NW = 32
LANES = 16
GROUPS = W // LANES
VCOV = 999936
QSZ = 1 << 18
QSH = 18
BLK = 4096


def _sc_gather(w128, tail, xp_flat, n, b, t):
    mesh = plsc.VectorSubcoreMesh(core_axis_name="core",
                                  subcore_axis_name="subcore")
    n_win = n // W
    per_w = n_win // NW
    n_tail = tail.shape[0]

    @functools.partial(
        pl.kernel,
        out_type=jax.ShapeDtypeStruct((t, DIM, b), jnp.float32),
        mesh=mesh,
        compiler_params=pltpu.CompilerParams(
            needs_layout_passes=False, disable_bounds_checks=True),
        scratch_types=[
            pltpu.VMEM((per_w * W,), jnp.int32),
            pltpu.VMEM((n_tail, DIM), jnp.float32),
            pltpu.VMEM((W,), jnp.int32),
            pltpu.VMEM((W,), jnp.int32),
            pltpu.VMEM((W, 128), jnp.float32),
            pltpu.VMEM((W, 128), jnp.float32),
            pltpu.VMEM((TPW, DIM, 128), jnp.float32),
            pltpu.VMEM((TPW, DIM, 128), jnp.float32),
            pltpu.SemaphoreType.DMA,
            pltpu.SemaphoreType.DMA,
            pltpu.SemaphoreType.DMA,
            pltpu.SemaphoreType.DMA,
        ],
    )
    def gather_kernel(w_hbm, tail_hbm, x_hbm, o_hbm, idx_all, tail_v,
                      q0, q1, g0, g1, t0b, t1b, gs0, gs1, os0, os1):
        wid = lax.axis_index("subcore") * 2 + lax.axis_index("core")
        base_win = wid * per_w
        pltpu.sync_copy(x_hbm.at[pl.ds(base_win * W, per_w * W)], idx_all)
        pltpu.sync_copy(tail_hbm, tail_v)

        qbufs = (q0, q1)
        gbufs = (g0, g1)
        tbufs = (t0b, t1b)
        gsems = (gs0, gs1)
        osems = (os0, os1)
        rows = [lax.iota(jnp.int32, LANES) + g * LANES
                for g in range(GROUPS)]

        def out_slice(k):
            win = base_win + k
            tt = win // 128
            bt = lax.rem(win, 128) // 4
            t0 = tt * 8 + lax.rem(win, 4) * TPW
            return o_hbm.at[pl.ds(t0, TPW), :, pl.ds(bt * 128, 128)]

        def prep_gather(k, p):
            @pl.loop(0, GROUPS)
            def _(j):
                xg = idx_all[pl.ds(k * W + j * LANES, LANES)]
                qbufs[p][pl.ds(j * LANES, LANES)] = xg & (QSZ - 1)

            pltpu.async_copy(w_hbm.at[qbufs[p]], gbufs[p], gsems[p])

        def wait_gather(p):
            pltpu.make_async_copy(w_hbm.at[qbufs[p]], gbufs[p],
                                  gsems[p]).wait()

        def start_out(k, p):
            pltpu.async_copy(tbufs[p], out_slice(k), osems[p])

        def wait_out(k, p):
            pltpu.make_async_copy(tbufs[p], out_slice(k), osems[p]).wait()

        def extract(k, p):
            gb = gbufs[p]
            tb = tbufs[p]
            cols, trows, tmask = [], [], []
            for g in range(GROUPS):
                xg = idx_all[pl.ds(k * W + g * LANES, LANES)]
                cols.append((xg >> QSH) * DIM)
                trows.append(jnp.maximum(xg - VCOV, 0))
                tmask.append(xg >= VCOV)

            @pl.loop(0, DIM)
            def _(d):
                dvec = jnp.full((LANES,), d, dtype=jnp.int32)
                vals = [plsc.load_gather(gb, [rows[g], cols[g] + d])
                        for g in range(GROUPS)]
                tvals = [plsc.load_gather(tail_v, [trows[g], dvec])
                         for g in range(GROUPS)]
                for g in range(GROUPS):
                    s, j = divmod(g, 128 // LANES)
                    tb[s, d, pl.ds(j * LANES, LANES)] = jnp.where(
                        tmask[g], tvals[g], vals[g])

        prep_gather(0, 0)

        @pl.loop(0, per_w, step=2)
        def _(k):
            for h in range(2):
                kk = k + h

                @pl.when(kk + 1 < per_w)
                def _():
                    prep_gather(kk + 1, (h + 1) % 2)

                wait_gather(h)

                @pl.when(kk >= 2)
                def _():
                    wait_out(kk - 2, h)

                extract(kk, h)
                start_out(kk, h)

        wait_out(per_w - 2, 0)
        wait_out(per_w - 1, 1)

    return gather_kernel(w128, tail, xp_flat)


def _tc_relayout(wt):
    nblk = QSZ // BLK
    full3 = (VCOV - 3 * QSZ) // BLK
    part3 = VCOV - 3 * QSZ - full3 * BLK

    def q3_copies(i, buf, sem):
        off = pl.multiple_of(i * BLK, 128)

        @pl.when(i < full3)
        def _():
            pltpu.make_async_copy(
                w_hbm_g[0].at[:, pl.ds(3 * QSZ + off, BLK)],
                buf.at[3], sem).start()

        @pl.when(i == full3)
        def _():
            pltpu.make_async_copy(
                w_hbm_g[0].at[:, pl.ds(3 * QSZ + full3 * BLK, part3)],
                buf.at[3, :, pl.ds(0, part3)], sem).start()

    def q3_waits(i, buf, sem):
        @pl.when(i < full3)
        def _():
            pltpu.make_async_copy(
                w_hbm_g[0].at[:, pl.ds(3 * QSZ, BLK)],
                buf.at[3], sem).wait()

        @pl.when(i == full3)
        def _():
            pltpu.make_async_copy(
                w_hbm_g[0].at[:, pl.ds(3 * QSZ, part3)],
                buf.at[3, :, pl.ds(0, part3)], sem).wait()

    w_hbm_g = []

    def start_block(i, buf, sem):
        off = pl.multiple_of(i * BLK, 128)
        for q in range(3):
            pltpu.make_async_copy(
                w_hbm_g[0].at[:, pl.ds(q * QSZ + off, BLK)],
                buf.at[q], sem).start()
        q3_copies(i, buf, sem)

    def wait_block(i, buf, sem):
        for q in range(3):
            pltpu.make_async_copy(
                w_hbm_g[0].at[:, pl.ds(q * QSZ, BLK)],
                buf.at[q], sem).wait()
        q3_waits(i, buf, sem)

    def body(w_hbm, o_ref, wbuf, sem0, sem1):
        if not w_hbm_g:
            w_hbm_g.append(w_hbm)
        else:
            w_hbm_g[0] = w_hbm
        i = pl.program_id(0)
        p = lax.rem(i, 2)
        sems = (sem0, sem1)

        @pl.when(i == 0)
        def _():
            start_block(i, wbuf.at[0], sem0)

        @pl.when((i + 1 < nblk) & (p == 0))
        def _():
            start_block(i + 1, wbuf.at[1], sem1)

        @pl.when((i + 1 < nblk) & (p == 1))
        def _():
            start_block(i + 1, wbuf.at[0], sem0)

        @pl.when(p == 0)
        def _():
            wait_block(i, wbuf.at[0], sem0)
            for q in range(4):
                o_ref[:, pl.ds(q * DIM, DIM)] = wbuf[0, q].T

        @pl.when(p == 1)
        def _():
            wait_block(i, wbuf.at[1], sem1)
            for q in range(4):
                o_ref[:, pl.ds(q * DIM, DIM)] = wbuf[1, q].T

    return pl.pallas_call(
        body,
        grid=(nblk,),
        in_specs=[pl.BlockSpec(memory_space=pltpu.MemorySpace.HBM)],
        out_specs=pl.BlockSpec((BLK, 128), lambda i: (i, 0)),
        out_shape=jax.ShapeDtypeStruct((QSZ, 128), jnp.float32),
        scratch_shapes=[pltpu.VMEM((2, 4, DIM, BLK), jnp.float32),
                        pltpu.SemaphoreType.DMA,
                        pltpu.SemaphoreType.DMA],
    )(wt)


def kernel(x, weight):
    b, t = x.shape
    n = x.size
    v = weight.shape[0]
    xp = x.reshape(b // 128, 128, t // 8, 8).transpose((2, 0, 3, 1))
    xp_flat = xp.reshape((n,)).astype(jnp.int32)
    w128 = _tc_relayout(weight.T)
    tail = lax.slice(weight, (VCOV, 0), (v, DIM))
    out3 = _sc_gather(w128, tail, xp_flat, n, b, t)
    return out3.transpose((2, 0, 1))
